# baseline (speedup 1.0000x reference)
import jax
import jax.numpy as jnp
from jax import lax
from jax.experimental import pallas as pl
from jax.experimental.pallas import tpu as pltpu
from jax.experimental.pallas import tpu_sc as plsc

_N = 10000
_E = 320000
_D = 128
_NC = 2
_NS = 16
_NW = _NC * _NS
_G = 72
_GPT = 140
_NGP = _NW * _GPT
_EP = _NGP * _G
_NB = 5
_RPT = 624
_REM = _N - _NS * _RPT
_LANES = 16


def _scale_group(rows, wv):
    @pl.loop(0, _G)
    def _(e):
        w16 = plsc.load_gather(wv, [jnp.full((_LANES,), 0, jnp.int32) + e])
        for r in range(_D // _LANES):
            sl = pl.ds(r * _LANES, _LANES)
            rows[e, sl] = rows[e, sl] * w16


def _sc_body(dst_hbm, src_hbm, w_hbm, x_hbm, out_hbm, acc, *bufs):
    bufs = [tuple(bufs[i * 8:(i + 1) * 8]) for i in range(_NB)]
    cid = lax.axis_index("c")
    sid = lax.axis_index("s")
    wid = sid * _NC + cid
    g0 = wid * _GPT

    zrows = bufs[0][3]

    @pl.loop(0, _G)
    def _(r):
        for d in range(0, _D, _LANES):
            zrows[r, pl.ds(d, _LANES)] = jnp.zeros((_LANES,), jnp.float32)

    base_row = sid * _RPT
    _NZ = _RPT // _G
    _TAIL = _RPT - _NZ * _G

    @pl.loop(0, _NZ)
    def _(i):
        pltpu.sync_copy(zrows, acc.at[pl.ds(base_row + i * _G, _G)])

    pltpu.sync_copy(zrows.at[pl.ds(0, _TAIL)],
                    acc.at[pl.ds(base_row + _NZ * _G, _TAIL)])

    @pl.when(sid == _NS - 1)
    def _():
        pltpu.sync_copy(zrows.at[pl.ds(0, _REM)],
                        acc.at[pl.ds(_NS * _RPT, _REM)])

    plsc.subcore_barrier()

    def start_fetch_src(j, b):
        (srci, dsti, wv, rows, sem_src, sem_dw, sem_g, sem_s) = b
        pltpu.async_copy(src_hbm.at[pl.ds((g0 + j) * _G, _G)], srci, sem_src)

    def wait_fetch_src(j, b):
        (srci, dsti, wv, rows, sem_src, sem_dw, sem_g, sem_s) = b
        pltpu.make_async_copy(src_hbm.at[pl.ds((g0 + j) * _G, _G)],
                              srci, sem_src).wait()

    def start_fetch_dw(j, b):
        (srci, dsti, wv, rows, sem_src, sem_dw, sem_g, sem_s) = b
        pltpu.async_copy(dst_hbm.at[pl.ds((g0 + j) * _G, _G)], dsti, sem_dw)
        pltpu.async_copy(w_hbm.at[pl.ds((g0 + j) * _G, _G)], wv, sem_dw)

    def wait_fetch_dw(j, b):
        (srci, dsti, wv, rows, sem_src, sem_dw, sem_g, sem_s) = b
        pltpu.make_async_copy(dst_hbm.at[pl.ds((g0 + j) * _G, _G)],
                              dsti, sem_dw).wait()
        pltpu.make_async_copy(w_hbm.at[pl.ds((g0 + j) * _G, _G)],
                              wv, sem_dw).wait()

    def start_gather(b):
        (srci, dsti, wv, rows, sem_src, sem_dw, sem_g, sem_s) = b
        pltpu.async_copy(x_hbm.at[srci], rows, sem_g)

    def wait_gather(b):
        (srci, dsti, wv, rows, sem_src, sem_dw, sem_g, sem_s) = b
        pltpu.make_async_copy(x_hbm.at[srci], rows, sem_g).wait()

    def start_scatter(b):
        (srci, dsti, wv, rows, sem_src, sem_dw, sem_g, sem_s) = b
        pltpu.async_copy(rows, acc.at[dsti], sem_s, add=True)

    def wait_scatter(b):
        (srci, dsti, wv, rows, sem_src, sem_dw, sem_g, sem_s) = b
        pltpu.make_async_copy(rows, acc.at[dsti], sem_s).wait()

    def phase(j, bufs_rot):
        bX, bN1, bN2, bN3, bN4 = bufs_rot

        @pl.when(j + 3 < _GPT)
        def _():
            wait_fetch_src(j + 3, bN3)
            start_gather(bN3)

        wait_gather(bX)

        @pl.when(j + 4 < _GPT)
        def _():
            start_fetch_src(j + 4, bN4)

        wait_fetch_dw(j, bX)
        _scale_group(bX[3], bX[2])
        start_scatter(bX)

        @pl.when(j > 0)
        def _():
            wait_scatter(bN4)

        @pl.when(j + 4 < _GPT)
        def _():
            start_fetch_dw(j + 4, bN4)

    for p in range(4):
        start_fetch_src(p, bufs[p])
        start_fetch_dw(p, bufs[p])
    for p in range(3):
        wait_fetch_src(p, bufs[p])
        start_gather(bufs[p])

    @pl.loop(0, _GPT, step=_NB)
    def _(i):
        for p in range(_NB):
            phase(i + p, tuple(bufs[(p + q) % _NB] for q in range(_NB)))

    wait_scatter(bufs[(_GPT - 1) % _NB])

    plsc.subcore_barrier()
    pltpu.sync_copy(acc.at[pl.ds(base_row, _RPT)],
                    out_hbm.at[cid, pl.ds(base_row, _RPT)])

    @pl.when(sid == _NS - 1)
    def _():
        pltpu.sync_copy(acc.at[pl.ds(_NS * _RPT, _REM)],
                        out_hbm.at[cid, pl.ds(_NS * _RPT, _REM)])


def _tc_add_body(p_ref, o_ref):
    o_ref[...] = p_ref[0] + p_ref[1]


def _combine_partials(partials):
    return pl.pallas_call(
        _tc_add_body,
        grid=(10,),
        in_specs=[pl.BlockSpec((2, _N // 10, _D), lambda i: (0, i, 0))],
        out_specs=pl.BlockSpec((_N // 10, _D), lambda i: (i, 0)),
        out_shape=jax.ShapeDtypeStruct((_N, _D), jnp.float32),
    )(partials)


@jax.jit
def kernel(t, x, edge_index, edge_weight):
    npad = _EP - _E
    pad_idx = (jnp.arange(npad, dtype=jnp.int32) * 37) % _N
    dst = jnp.concatenate([edge_index[0], pad_idx])
    src = jnp.concatenate([edge_index[1], pad_idx])
    w = jnp.concatenate([edge_weight, jnp.zeros((npad,), jnp.float32)])

    mesh = plsc.VectorSubcoreMesh(core_axis_name="c", subcore_axis_name="s")
    buf_types = []
    for _ in range(_NB):
        buf_types += [
            pltpu.VMEM((_G,), jnp.int32),
            pltpu.VMEM((_G,), jnp.int32),
            pltpu.VMEM((_G,), jnp.float32),
            pltpu.VMEM((_G, _D), jnp.float32),
            pltpu.SemaphoreType.DMA,
            pltpu.SemaphoreType.DMA,
            pltpu.SemaphoreType.DMA,
            pltpu.SemaphoreType.DMA,
        ]
    spmm = pl.kernel(
        _sc_body,
        out_type=jax.ShapeDtypeStruct((_NC, _N, _D), jnp.float32),
        mesh=mesh,
        compiler_params=pltpu.CompilerParams(needs_layout_passes=False),
        scratch_types=[pltpu.VMEM_SHARED((_N, _D), jnp.float32)] + buf_types,
    )
    partials = spmm(dst, src, w, x)
    return _combine_partials(partials)


# speedup vs baseline: 1.0003x; 1.0003x over previous
import jax
import jax.numpy as jnp
from jax import lax
from jax.experimental import pallas as pl
from jax.experimental.pallas import tpu as pltpu
from jax.experimental.pallas import tpu_sc as plsc

_N = 10000
_E = 320000
_D = 128
_NC = 2
_NS = 16
_NW = _NC * _NS
_G = 72
_GPT = 140
_NGP = _NW * _GPT
_EP = _NGP * _G
_NB = 5
_RPT = 624
_REM = _N - _NS * _RPT
_LANES = 16


def _scale_group(rows, wv):
    @pl.loop(0, _G)
    def _(e):
        w16 = plsc.load_gather(wv, [jnp.full((_LANES,), 0, jnp.int32) + e])
        for r in range(_D // _LANES):
            sl = pl.ds(r * _LANES, _LANES)
            rows[e, sl] = rows[e, sl] * w16


def _sc_body(dst_hbm, src_hbm, w_hbm, x_hbm, out_hbm, acc, *bufs):
    bufs = [tuple(bufs[i * 8:(i + 1) * 8]) for i in range(_NB)]
    cid = lax.axis_index("c")
    sid = lax.axis_index("s")
    wid = sid * _NC + cid
    g0 = wid * _GPT

    zrows = bufs[0][3]

    @pl.loop(0, _G)
    def _(r):
        for d in range(0, _D, _LANES):
            zrows[r, pl.ds(d, _LANES)] = jnp.zeros((_LANES,), jnp.float32)

    base_row = sid * _RPT
    _NZ = _RPT // _G
    _TAIL = _RPT - _NZ * _G

    @pl.loop(0, _NZ)
    def _(i):
        pltpu.sync_copy(zrows, acc.at[pl.ds(base_row + i * _G, _G)])

    pltpu.sync_copy(zrows.at[pl.ds(0, _TAIL)],
                    acc.at[pl.ds(base_row + _NZ * _G, _TAIL)])

    @pl.when(sid == _NS - 1)
    def _():
        pltpu.sync_copy(zrows.at[pl.ds(0, _REM)],
                        acc.at[pl.ds(_NS * _RPT, _REM)])

    plsc.subcore_barrier()

    def start_fetch_src(j, b):
        (srci, dsti, wv, rows, sem_src, sem_dw, sem_g, sem_s) = b
        pltpu.async_copy(src_hbm.at[pl.ds((g0 + j) * _G, _G)], srci, sem_src)

    def wait_fetch_src(j, b):
        (srci, dsti, wv, rows, sem_src, sem_dw, sem_g, sem_s) = b
        pltpu.make_async_copy(src_hbm.at[pl.ds((g0 + j) * _G, _G)],
                              srci, sem_src).wait()

    def start_fetch_dw(j, b):
        (srci, dsti, wv, rows, sem_src, sem_dw, sem_g, sem_s) = b
        pltpu.async_copy(dst_hbm.at[pl.ds((g0 + j) * _G, _G)], dsti, sem_dw)
        pltpu.async_copy(w_hbm.at[pl.ds((g0 + j) * _G, _G)], wv, sem_dw)

    def wait_fetch_dw(j, b):
        (srci, dsti, wv, rows, sem_src, sem_dw, sem_g, sem_s) = b
        pltpu.make_async_copy(dst_hbm.at[pl.ds((g0 + j) * _G, _G)],
                              dsti, sem_dw).wait()
        pltpu.make_async_copy(w_hbm.at[pl.ds((g0 + j) * _G, _G)],
                              wv, sem_dw).wait()

    def start_gather(b):
        (srci, dsti, wv, rows, sem_src, sem_dw, sem_g, sem_s) = b
        pltpu.async_copy(x_hbm.at[srci.at[pl.ds(0, 40)]],
                         rows.at[pl.ds(0, 40)], sem_g)
        pltpu.async_copy(x_hbm.at[srci.at[pl.ds(40, 32)]],
                         rows.at[pl.ds(40, 32)], sem_g)

    def wait_gather(b):
        (srci, dsti, wv, rows, sem_src, sem_dw, sem_g, sem_s) = b
        pltpu.make_async_copy(x_hbm.at[srci.at[pl.ds(0, 40)]],
                              rows.at[pl.ds(0, 40)], sem_g).wait()
        pltpu.make_async_copy(x_hbm.at[srci.at[pl.ds(40, 32)]],
                              rows.at[pl.ds(40, 32)], sem_g).wait()

    def start_scatter(b):
        (srci, dsti, wv, rows, sem_src, sem_dw, sem_g, sem_s) = b
        pltpu.async_copy(rows, acc.at[dsti], sem_s, add=True)

    def wait_scatter(b):
        (srci, dsti, wv, rows, sem_src, sem_dw, sem_g, sem_s) = b
        pltpu.make_async_copy(rows, acc.at[dsti], sem_s).wait()

    def phase(j, bufs_rot):
        bX, bN1, bN2, bN3, bN4 = bufs_rot

        @pl.when(j + 3 < _GPT)
        def _():
            wait_fetch_src(j + 3, bN3)
            start_gather(bN3)

        wait_gather(bX)

        @pl.when(j + 4 < _GPT)
        def _():
            start_fetch_src(j + 4, bN4)

        wait_fetch_dw(j, bX)
        _scale_group(bX[3], bX[2])
        start_scatter(bX)

        @pl.when(j > 0)
        def _():
            wait_scatter(bN4)

        @pl.when(j + 4 < _GPT)
        def _():
            start_fetch_dw(j + 4, bN4)

    for p in range(4):
        start_fetch_src(p, bufs[p])
        start_fetch_dw(p, bufs[p])
    for p in range(3):
        wait_fetch_src(p, bufs[p])
        start_gather(bufs[p])

    @pl.loop(0, _GPT, step=_NB)
    def _(i):
        for p in range(_NB):
            phase(i + p, tuple(bufs[(p + q) % _NB] for q in range(_NB)))

    wait_scatter(bufs[(_GPT - 1) % _NB])

    plsc.subcore_barrier()
    pltpu.sync_copy(acc.at[pl.ds(base_row, _RPT)],
                    out_hbm.at[cid, pl.ds(base_row, _RPT)])

    @pl.when(sid == _NS - 1)
    def _():
        pltpu.sync_copy(acc.at[pl.ds(_NS * _RPT, _REM)],
                        out_hbm.at[cid, pl.ds(_NS * _RPT, _REM)])


def _tc_add_body(p_ref, o_ref):
    o_ref[...] = p_ref[0] + p_ref[1]


def _combine_partials(partials):
    return pl.pallas_call(
        _tc_add_body,
        grid=(10,),
        in_specs=[pl.BlockSpec((2, _N // 10, _D), lambda i: (0, i, 0))],
        out_specs=pl.BlockSpec((_N // 10, _D), lambda i: (i, 0)),
        out_shape=jax.ShapeDtypeStruct((_N, _D), jnp.float32),
    )(partials)


@jax.jit
def kernel(t, x, edge_index, edge_weight):
    npad = _EP - _E
    pad_idx = (jnp.arange(npad, dtype=jnp.int32) * 37) % _N
    dst = jnp.concatenate([edge_index[0], pad_idx])
    src = jnp.concatenate([edge_index[1], pad_idx])
    w = jnp.concatenate([edge_weight, jnp.zeros((npad,), jnp.float32)])

    mesh = plsc.VectorSubcoreMesh(core_axis_name="c", subcore_axis_name="s")
    buf_types = []
    for _ in range(_NB):
        buf_types += [
            pltpu.VMEM((_G,), jnp.int32),
            pltpu.VMEM((_G,), jnp.int32),
            pltpu.VMEM((_G,), jnp.float32),
            pltpu.VMEM((_G, _D), jnp.float32),
            pltpu.SemaphoreType.DMA,
            pltpu.SemaphoreType.DMA,
            pltpu.SemaphoreType.DMA,
            pltpu.SemaphoreType.DMA,
        ]
    spmm = pl.kernel(
        _sc_body,
        out_type=jax.ShapeDtypeStruct((_NC, _N, _D), jnp.float32),
        mesh=mesh,
        compiler_params=pltpu.CompilerParams(needs_layout_passes=False),
        scratch_types=[pltpu.VMEM_SHARED((_N, _D), jnp.float32)] + buf_types,
    )
    partials = spmm(dst, src, w, x)
    return _combine_partials(partials)


# speedup vs baseline: 1.0138x; 1.0134x over previous
import jax
import jax.numpy as jnp
from jax import lax
from jax.experimental import pallas as pl
from jax.experimental.pallas import tpu as pltpu
from jax.experimental.pallas import tpu_sc as plsc

_N = 10000
_E = 320000
_D = 128
_NC = 2
_NS = 16
_NW = _NC * _NS
_G = 72
_GPT = 140
_NGP = _NW * _GPT
_EP = _NGP * _G
_NB = 5
_RPT = 624
_REM = _N - _NS * _RPT
_LANES = 16


def _scale_group(rows, wv):
    @pl.loop(0, _G)
    def _(e):
        w16 = plsc.load_gather(wv, [jnp.full((_LANES,), 0, jnp.int32) + e])
        for r in range(_D // _LANES):
            sl = pl.ds(r * _LANES, _LANES)
            rows[e, sl] = rows[e, sl] * w16


def _sc_body(dst_hbm, src_hbm, w_hbm, x_hbm, out_hbm, acc, *bufs):
    bufs = [tuple(bufs[i * 8:(i + 1) * 8]) for i in range(_NB)]
    cid = lax.axis_index("c")
    sid = lax.axis_index("s")
    wid = sid * _NC + cid
    g0 = wid * _GPT

    zrows = bufs[4][3]


    def start_fetch_src(j, b):
        (srci, dsti, wv, rows, sem_src, sem_dw, sem_g, sem_s) = b
        pltpu.async_copy(src_hbm.at[pl.ds((g0 + j) * _G, _G)], srci, sem_src)

    def wait_fetch_src(j, b):
        (srci, dsti, wv, rows, sem_src, sem_dw, sem_g, sem_s) = b
        pltpu.make_async_copy(src_hbm.at[pl.ds((g0 + j) * _G, _G)],
                              srci, sem_src).wait()

    def start_fetch_dw(j, b):
        (srci, dsti, wv, rows, sem_src, sem_dw, sem_g, sem_s) = b
        pltpu.async_copy(dst_hbm.at[pl.ds((g0 + j) * _G, _G)], dsti, sem_dw)
        pltpu.async_copy(w_hbm.at[pl.ds((g0 + j) * _G, _G)], wv, sem_dw)

    def wait_fetch_dw(j, b):
        (srci, dsti, wv, rows, sem_src, sem_dw, sem_g, sem_s) = b
        pltpu.make_async_copy(dst_hbm.at[pl.ds((g0 + j) * _G, _G)],
                              dsti, sem_dw).wait()
        pltpu.make_async_copy(w_hbm.at[pl.ds((g0 + j) * _G, _G)],
                              wv, sem_dw).wait()

    def start_gather(b):
        (srci, dsti, wv, rows, sem_src, sem_dw, sem_g, sem_s) = b
        pltpu.async_copy(x_hbm.at[srci], rows, sem_g)

    def wait_gather(b):
        (srci, dsti, wv, rows, sem_src, sem_dw, sem_g, sem_s) = b
        pltpu.make_async_copy(x_hbm.at[srci], rows, sem_g).wait()

    def start_scatter(b):
        (srci, dsti, wv, rows, sem_src, sem_dw, sem_g, sem_s) = b
        pltpu.async_copy(rows, acc.at[dsti], sem_s, add=True)

    def wait_scatter(b):
        (srci, dsti, wv, rows, sem_src, sem_dw, sem_g, sem_s) = b
        pltpu.make_async_copy(rows, acc.at[dsti], sem_s).wait()

    def phase(j, bufs_rot):
        bX, bN1, bN2, bN3, bN4 = bufs_rot

        @pl.when(j + 3 < _GPT)
        def _():
            wait_fetch_src(j + 3, bN3)
            start_gather(bN3)

        wait_gather(bX)

        @pl.when(j + 4 < _GPT)
        def _():
            start_fetch_src(j + 4, bN4)

        wait_fetch_dw(j, bX)
        _scale_group(bX[3], bX[2])
        start_scatter(bX)

        @pl.when(j > 0)
        def _():
            wait_scatter(bN4)

        @pl.when(j + 4 < _GPT)
        def _():
            start_fetch_dw(j + 4, bN4)


    for p in range(4):
        start_fetch_src(p, bufs[p])
        start_fetch_dw(p, bufs[p])
    for p in range(3):
        wait_fetch_src(p, bufs[p])
        start_gather(bufs[p])

    @pl.loop(0, _G)
    def _(r):
        for d in range(0, _D, _LANES):
            zrows[r, pl.ds(d, _LANES)] = jnp.zeros((_LANES,), jnp.float32)

    base_row = sid * _RPT
    _NZ = _RPT // _G
    _TAIL = _RPT - _NZ * _G

    @pl.loop(0, _NZ)
    def _(i):
        pltpu.sync_copy(zrows, acc.at[pl.ds(base_row + i * _G, _G)])

    pltpu.sync_copy(zrows.at[pl.ds(0, _TAIL)],
                    acc.at[pl.ds(base_row + _NZ * _G, _TAIL)])

    @pl.when(sid == _NS - 1)
    def _():
        pltpu.sync_copy(zrows.at[pl.ds(0, _REM)],
                        acc.at[pl.ds(_NS * _RPT, _REM)])

    plsc.subcore_barrier()

    @pl.loop(0, _GPT, step=_NB)
    def _(i):
        for p in range(_NB):
            phase(i + p, tuple(bufs[(p + q) % _NB] for q in range(_NB)))

    wait_scatter(bufs[(_GPT - 1) % _NB])

    plsc.subcore_barrier()
    pltpu.sync_copy(acc.at[pl.ds(base_row, _RPT)],
                    out_hbm.at[cid, pl.ds(base_row, _RPT)])

    @pl.when(sid == _NS - 1)
    def _():
        pltpu.sync_copy(acc.at[pl.ds(_NS * _RPT, _REM)],
                        out_hbm.at[cid, pl.ds(_NS * _RPT, _REM)])


def _tc_add_body(p_ref, o_ref):
    o_ref[...] = p_ref[0] + p_ref[1]


def _combine_partials(partials):
    return pl.pallas_call(
        _tc_add_body,
        grid=(10,),
        in_specs=[pl.BlockSpec((2, _N // 10, _D), lambda i: (0, i, 0))],
        out_specs=pl.BlockSpec((_N // 10, _D), lambda i: (i, 0)),
        out_shape=jax.ShapeDtypeStruct((_N, _D), jnp.float32),
    )(partials)


@jax.jit
def kernel(t, x, edge_index, edge_weight):
    npad = _EP - _E
    pad_idx = (jnp.arange(npad, dtype=jnp.int32) * 37) % _N
    dst = jnp.concatenate([edge_index[0], pad_idx])
    src = jnp.concatenate([edge_index[1], pad_idx])
    w = jnp.concatenate([edge_weight, jnp.zeros((npad,), jnp.float32)])

    mesh = plsc.VectorSubcoreMesh(core_axis_name="c", subcore_axis_name="s")
    buf_types = []
    for _ in range(_NB):
        buf_types += [
            pltpu.VMEM((_G,), jnp.int32),
            pltpu.VMEM((_G,), jnp.int32),
            pltpu.VMEM((_G,), jnp.float32),
            pltpu.VMEM((_G, _D), jnp.float32),
            pltpu.SemaphoreType.DMA,
            pltpu.SemaphoreType.DMA,
            pltpu.SemaphoreType.DMA,
            pltpu.SemaphoreType.DMA,
        ]
    spmm = pl.kernel(
        _sc_body,
        out_type=jax.ShapeDtypeStruct((_NC, _N, _D), jnp.float32),
        mesh=mesh,
        compiler_params=pltpu.CompilerParams(needs_layout_passes=False),
        scratch_types=[pltpu.VMEM_SHARED((_N, _D), jnp.float32)] + buf_types,
    )
    partials = spmm(dst, src, w, x)
    return _combine_partials(partials)
